# Initial kernel scaffold; baseline (speedup 1.0000x reference)
import jax
import jax.numpy as jnp
from jax.experimental import pallas as pl


def kernel(h, edge_index, WQ, WK, WV, WG):
    raise NotImplementedError("write your pallas kernel here")


# speedup vs baseline: 3.5464x; 3.5464x over previous
import jax
import jax.numpy as jnp
from jax import lax
from jax.experimental import pallas as pl
from jax.experimental.pallas import tpu as pltpu
from jax.experimental.pallas import tpu_sc as plsc

N = 50000
E = 800000
IN_DIM = 128
D = 64

NC, NS = 2, 16
NW = NC * NS
NV = 50400
EP = 819200
ROWF = 80
NACC = 50176
TPR = NACC // NS
CH = 128
SB = 1024
RB = 112

_SC_PARAMS = pltpu.CompilerParams(
    needs_layout_passes=False, use_tc_tiling_on_sc=False)


def _rsqrt(x):
    i = lax.bitcast_convert_type(x, jnp.int32)
    i = jnp.int32(0x5F3759DF) - lax.shift_right_arithmetic(i, 1)
    y = lax.bitcast_convert_type(i, jnp.float32)
    for _ in range(3):
        y = y * (1.5 - 0.5 * x * y * y)
    return y


def _proj_body(h_ref, wq_ref, wk_ref, wv_ref, wg_ref,
               a_ref, b_ref, v0_ref, v1_ref):
    hb = h_ref[...]
    kb = jnp.dot(hb, wk_ref[...], preferred_element_type=jnp.float32)
    qb = jnp.dot(hb, wq_ref[...], preferred_element_type=jnp.float32)
    vb = jnp.dot(hb, wv_ref[...], preferred_element_type=jnp.float32)
    gb = jnp.dot(hb, wg_ref[...], preferred_element_type=jnp.float32)
    a_ref[...] = jnp.concatenate([kb, gb], axis=1)
    b_ref[...] = jnp.concatenate([qb, gb], axis=1)
    v0_ref[...] = vb[:, :32]
    v1_ref[...] = vb[:, 32:]


def _proj(hp, wq, wk, wv, wgp):
    rb = 400
    return pl.pallas_call(
        _proj_body,
        grid=(NV // rb,),
        in_specs=[
            pl.BlockSpec((rb, IN_DIM), lambda i: (i, 0)),
            pl.BlockSpec((IN_DIM, D), lambda i: (0, 0)),
            pl.BlockSpec((IN_DIM, D), lambda i: (0, 0)),
            pl.BlockSpec((IN_DIM, D), lambda i: (0, 0)),
            pl.BlockSpec((IN_DIM, 16), lambda i: (0, 0)),
        ],
        out_specs=[
            pl.BlockSpec((rb, ROWF), lambda i: (i, 0)),
            pl.BlockSpec((rb, ROWF), lambda i: (i, 0)),
            pl.BlockSpec((rb, 32), lambda i: (i, 0)),
            pl.BlockSpec((rb, 32), lambda i: (i, 0)),
        ],
        out_shape=[
            jax.ShapeDtypeStruct((NV, ROWF), jnp.float32),
            jax.ShapeDtypeStruct((NV, ROWF), jnp.float32),
            jax.ShapeDtypeStruct((NV, 32), jnp.float32),
            jax.ShapeDtypeStruct((NV, 32), jnp.float32),
        ],
    )(hp, wq, wk, wv, wgp)


def _edge_weight_body(a_hbm, b_hbm, src_hbm, dst_hbm, zeros_z,
                      w_hbm, zpart_hbm,
                      sidx, didx, a_v, b_v, wbuf, val_z, ezbuf, zbuf,
                      acc_z, sem):
    c = lax.axis_index("c")
    s = lax.axis_index("s")
    wid = s * NC + c
    per_tile = EP // NW
    n_super = per_tile // SB
    lanes = lax.iota(jnp.int32, 16)

    pltpu.sync_copy(zeros_z, acc_z.at[pl.ds(pl.multiple_of(s * TPR, 8), TPR)])
    plsc.subcore_barrier()

    def superblock(sb_i, _):
        sb_base = pl.multiple_of(wid * per_tile + sb_i * SB, SB)
        row0 = pl.multiple_of(sb_base // CH, 8)
        pltpu.sync_copy(src_hbm.at[pl.ds(row0, SB // CH)], sidx)
        pltpu.sync_copy(dst_hbm.at[pl.ds(row0, SB // CH)], didx)

        def chunk(k, _):
            pltpu.async_copy(a_hbm.at[sidx.at[k]], a_v, sem).wait()
            pltpu.async_copy(b_hbm.at[didx.at[k]], b_v, sem).wait()

            def group(g, _):
                svs, g2s = [], []
                for e in range(16):
                    r = g * 16 + e
                    acc = (a_v[r, pl.ds(0, 16)] * b_v[r, pl.ds(0, 16)]
                           + a_v[r, pl.ds(16, 16)] * b_v[r, pl.ds(16, 16)]
                           + a_v[r, pl.ds(32, 16)] * b_v[r, pl.ds(32, 16)]
                           + a_v[r, pl.ds(48, 16)] * b_v[r, pl.ds(48, 16)])
                    gd = a_v[r, pl.ds(64, 16)] - b_v[r, pl.ds(64, 16)]
                    svs.append(jnp.where(lanes == e, jnp.sum(acc), 0.0))
                    g2s.append(jnp.where(lanes == e, jnp.sum(gd * gd), 0.0))
                while len(svs) > 1:
                    svs = [x + y for x, y in zip(svs[::2], svs[1::2])]
                    g2s = [x + y for x, y in zip(g2s[::2], g2s[1::2])]
                sv, g2 = svs[0], g2s[0] + 1e-6
                score = jnp.exp(jnp.clip(sv * 0.125, -5.0, 5.0))
                dist = -(g2 * _rsqrt(g2))
                distance = jnp.exp(jnp.clip(dist * 0.125, -5.0, 5.0))
                wbuf[pl.ds(k * CH + g * 16, 16)] = score * distance
                for e in range(16):
                    val_z[g * 16 + e, pl.ds(0, 16)] = jnp.where(
                        lanes == 0, score[e], 0.0)
                return 0

            lax.fori_loop(0, CH // 16, group, 0)
            pltpu.sync_copy(val_z, acc_z.at[didx.at[k]], add=True)
            return 0

        lax.fori_loop(0, SB // CH, chunk, 0)
        pltpu.sync_copy(wbuf, w_hbm.at[pl.ds(sb_base, SB)])
        return 0

    lax.fori_loop(0, n_super, superblock, 0)
    plsc.subcore_barrier()

    def zchunk(j, _):
        r0 = pl.multiple_of(s * TPR + j * RB, 8)
        pltpu.sync_copy(acc_z.at[pl.ds(r0, RB)], ezbuf)

        def t16(t, _):
            zv = jnp.zeros((16,), jnp.float32)
            for e in range(16):
                zv = jnp.where(lanes == e, ezbuf[t * 16 + e, pl.ds(0, 16)][0], zv)
            zbuf[pl.ds(t * 16, 16)] = zv
            return 0

        lax.fori_loop(0, RB // 16, t16, 0)
        pltpu.sync_copy(zbuf, zpart_hbm.at[c, pl.ds(r0, RB)])
        return 0

    lax.fori_loop(0, TPR // RB, zchunk, 0)


def _edge_weights(a, b, src2d, dst2d, zeros_z):
    mesh = plsc.VectorSubcoreMesh(core_axis_name="c", subcore_axis_name="s")
    f = pl.kernel(
        _edge_weight_body,
        out_type=[
            jax.ShapeDtypeStruct((EP,), jnp.float32),
            jax.ShapeDtypeStruct((NC, NACC), jnp.float32),
        ],
        mesh=mesh,
        scratch_types=[
            pltpu.VMEM((SB // CH, CH), jnp.int32),
            pltpu.VMEM((SB // CH, CH), jnp.int32),
            pltpu.VMEM((CH, ROWF), jnp.float32),
            pltpu.VMEM((CH, ROWF), jnp.float32),
            pltpu.VMEM((SB,), jnp.float32),
            pltpu.VMEM((CH, 16), jnp.float32),
            pltpu.VMEM((RB, 16), jnp.float32),
            pltpu.VMEM((RB,), jnp.float32),
            pltpu.VMEM_SHARED((NACC, 16), jnp.float32),
            pltpu.SemaphoreType.DMA,
        ],
        compiler_params=_SC_PARAMS,
    )
    return f(a, b, src2d, dst2d, zeros_z)


def _agg_body(vtab_hbm, src_hbm, dst_hbm, w_hbm, zpart_hbm, zeros_hbm,
              out_hbm,
              sidx, didx, vidx, w_v, v_v, val_v, evbuf, ovbuf, zb0, zb1,
              acc, sem):
    c = lax.axis_index("c")
    s = lax.axis_index("s")
    per_tile = EP // NS
    n_super = per_tile // SB
    voff = c * NV
    lanes = lax.iota(jnp.int32, 16)

    pltpu.sync_copy(zeros_hbm, acc.at[pl.ds(pl.multiple_of(s * TPR, 8), TPR)])
    plsc.subcore_barrier()

    def superblock(sb_i, _):
        sb_base = pl.multiple_of(s * per_tile + sb_i * SB, SB)
        row0 = pl.multiple_of(sb_base // CH, 8)
        pltpu.sync_copy(src_hbm.at[pl.ds(row0, SB // CH)], sidx)
        pltpu.sync_copy(dst_hbm.at[pl.ds(row0, SB // CH)], didx)
        pltpu.sync_copy(w_hbm.at[pl.ds(sb_base, SB)], w_v)

        def chunk(k, _):
            for j in range(CH // 16):
                vidx[pl.ds(j * 16, 16)] = sidx[k, pl.ds(j * 16, 16)] + voff
            pltpu.async_copy(vtab_hbm.at[vidx], v_v, sem).wait()

            def group(g, _):
                base16 = k * CH + g * 16
                w16 = w_v[pl.ds(base16, 16)]
                for e in range(16):
                    r = g * 16 + e
                    w = w16[e]
                    val_v[r, pl.ds(0, 16)] = v_v[r, pl.ds(0, 16)] * w
                    val_v[r, pl.ds(16, 16)] = v_v[r, pl.ds(16, 16)] * w
                return 0

            lax.fori_loop(0, CH // 16, group, 0)
            pltpu.sync_copy(val_v, acc.at[didx.at[k]], add=True)
            return 0

        lax.fori_loop(0, SB // CH, chunk, 0)
        return 0

    lax.fori_loop(0, n_super, superblock, 0)
    plsc.subcore_barrier()

    def ep_chunk(j, _):
        r0 = pl.multiple_of(s * TPR + j * RB, 8)
        pltpu.sync_copy(acc.at[pl.ds(r0, RB)], evbuf)
        pltpu.sync_copy(zpart_hbm.at[0, pl.ds(r0, RB)], zb0)
        pltpu.sync_copy(zpart_hbm.at[1, pl.ds(r0, RB)], zb1)

        def t16(t, _):
            z16 = zb0[pl.ds(t * 16, 16)] + zb1[pl.ds(t * 16, 16)]
            inv = 1.0 / jnp.where(z16 > 0.0, z16, 1.0)
            for e in range(16):
                r = t * 16 + e
                iv = inv[e]
                ovbuf[r, pl.ds(0, 16)] = evbuf[r, pl.ds(0, 16)] * iv
                ovbuf[r, pl.ds(16, 16)] = evbuf[r, pl.ds(16, 16)] * iv
            return 0

        lax.fori_loop(0, RB // 16, t16, 0)

        @pl.when(c == 0)
        def _():
            pltpu.sync_copy(ovbuf, out_hbm.at[pl.ds(r0, RB), pl.ds(0, 32)])

        @pl.when(c == 1)
        def _():
            pltpu.sync_copy(ovbuf, out_hbm.at[pl.ds(r0, RB), pl.ds(32, 32)])
        return 0

    lax.fori_loop(0, TPR // RB, ep_chunk, 0)


def _aggregate(vtab, src2d, dst2d, w, zpart, zeros_acc):
    mesh = plsc.VectorSubcoreMesh(core_axis_name="c", subcore_axis_name="s")
    f = pl.kernel(
        _agg_body,
        out_type=jax.ShapeDtypeStruct((NACC, D), jnp.float32),
        mesh=mesh,
        scratch_types=[
            pltpu.VMEM((SB // CH, CH), jnp.int32),
            pltpu.VMEM((SB // CH, CH), jnp.int32),
            pltpu.VMEM((CH,), jnp.int32),
            pltpu.VMEM((SB,), jnp.float32),
            pltpu.VMEM((CH, 32), jnp.float32),
            pltpu.VMEM((CH, 32), jnp.float32),
            pltpu.VMEM((RB, 32), jnp.float32),
            pltpu.VMEM((RB, 32), jnp.float32),
            pltpu.VMEM((RB,), jnp.float32),
            pltpu.VMEM((RB,), jnp.float32),
            pltpu.VMEM_SHARED((NACC, 32), jnp.float32),
            pltpu.SemaphoreType.DMA,
        ],
        compiler_params=_SC_PARAMS,
    )
    return f(vtab, src2d, dst2d, w, zpart, zeros_acc)


@jax.jit
def kernel(h, edge_index, WQ, WK, WV, WG):
    hp = jnp.zeros((NV, IN_DIM), jnp.float32).at[:N].set(h)
    wgp = jnp.zeros((IN_DIM, 16), jnp.float32).at[:, :3].set(WG)
    a, b, v0, v1 = _proj(hp, WQ, WK, WV, wgp)
    vtab = jnp.concatenate([v0, v1], axis=0)

    pad = jnp.full((EP - E,), N, jnp.int32)
    src2d = jnp.concatenate([edge_index[0], pad]).reshape(EP // CH, CH)
    dst2d = jnp.concatenate([edge_index[1], pad]).reshape(EP // CH, CH)

    zeros_z = jnp.zeros((TPR, 16), jnp.float32)
    w, zpart = _edge_weights(a, b, src2d, dst2d, zeros_z)
    zeros_acc = jnp.zeros((TPR, 32), jnp.float32)
    outf = _aggregate(vtab, src2d, dst2d, w, zpart, zeros_acc)
    return outf[:N].reshape(N, 1, D)
